# baseline (speedup 1.0000x reference)
import functools

import jax
import jax.numpy as jnp
from jax import lax
from jax.experimental import pallas as pl
from jax.experimental.pallas import tpu as pltpu
from jax.experimental.pallas import tpu_sc as plsc

BATCH = 16384
DIM = 64
NC = 2
NS = 16
NW = NC * NS
PER_W = BATCH // NW


def _sc_gather_body(user_hbm, item_hbm, gu_t, gi_t, mu_t, mi_t,
                    gu_o, gi_o, mu_o, mi_o,
                    uidx, iidx, bufa, bufb, sema, semb):
    wid = lax.axis_index("s") * NC + lax.axis_index("c")
    base = wid * PER_W
    pltpu.sync_copy(user_hbm.at[pl.ds(base, PER_W)], uidx)
    pltpu.sync_copy(item_hbm.at[pl.ds(base, PER_W)], iidx)
    cpa = pltpu.async_copy(gu_t.at[uidx], bufa, sema)
    cpb = pltpu.async_copy(gi_t.at[iidx], bufb, semb)
    cpa.wait()
    pltpu.sync_copy(bufa, gu_o.at[pl.ds(base, PER_W)])
    cpa2 = pltpu.async_copy(mu_t.at[uidx], bufa, sema)
    cpb.wait()
    pltpu.sync_copy(bufb, gi_o.at[pl.ds(base, PER_W)])
    cpb2 = pltpu.async_copy(mi_t.at[iidx], bufb, semb)
    cpa2.wait()
    pltpu.sync_copy(bufa, mu_o.at[pl.ds(base, PER_W)])
    cpb2.wait()
    pltpu.sync_copy(bufb, mi_o.at[pl.ds(base, PER_W)])


def _make_sc_gather():
    mesh = plsc.VectorSubcoreMesh(core_axis_name="c", subcore_axis_name="s")
    row = jax.ShapeDtypeStruct((BATCH, DIM), jnp.float32)
    return pl.kernel(
        _sc_gather_body,
        out_type=[row, row, row, row],
        mesh=mesh,
        scratch_types=[
            pltpu.VMEM((PER_W,), jnp.int32),
            pltpu.VMEM((PER_W,), jnp.int32),
            pltpu.VMEM((PER_W, DIM), jnp.float32),
            pltpu.VMEM((PER_W, DIM), jnp.float32),
            pltpu.SemaphoreType.DMA,
            pltpu.SemaphoreType.DMA,
        ],
        compiler_params=pltpu.CompilerParams(use_tc_tiling_on_sc=False),
    )


BLK = 512


def _tc_dense_body(gu, gi, mu, mi, w0, b0, w1, b1, hw, hb, nw, nb,
                   fused_o, score_o):
    mlp_x = jnp.concatenate([mu[...], mi[...]], axis=1)
    h = jnp.maximum(jnp.dot(mlp_x, w0[...],
                            preferred_element_type=jnp.float32) + b0[...], 0.0)
    mlp_out = jnp.maximum(jnp.dot(h, w1[...],
                                  preferred_element_type=jnp.float32) + b1[...], 0.0)
    gmf = gu[...] * gi[...]
    fused_in = jnp.concatenate([0.5 * gmf, 0.5 * mlp_out], axis=1)
    fused = jnp.dot(fused_in, hw[...],
                    preferred_element_type=jnp.float32) + hb[...]
    fused_o[...] = fused
    score_o[...] = jnp.sum(fused * nw[...], axis=1, keepdims=True) + nb[...]


def _make_tc_dense():
    grid = (BATCH // BLK,)
    blk_in = pl.BlockSpec((BLK, DIM), lambda i: (i, 0))
    full = lambda shape: pl.BlockSpec(shape, lambda i: (0, 0))
    return pl.pallas_call(
        _tc_dense_body,
        grid=grid,
        in_specs=[
            blk_in, blk_in, blk_in, blk_in,
            full((2 * DIM, 2 * DIM)),
            full((1, 2 * DIM)),
            full((2 * DIM, DIM)),
            full((1, DIM)),
            full((2 * DIM, DIM)),
            full((1, DIM)),
            full((1, DIM)),
            full((1, 1)),
        ],
        out_specs=[
            pl.BlockSpec((BLK, DIM), lambda i: (i, 0)),
            pl.BlockSpec((BLK, 1), lambda i: (i, 0)),
        ],
        out_shape=[
            jax.ShapeDtypeStruct((BATCH, DIM), jnp.float32),
            jax.ShapeDtypeStruct((BATCH, 1), jnp.float32),
        ],
    )


def kernel(user, item, gmf_user_table, gmf_item_table, mlp_user_table,
           mlp_item_table, mlp_W0, mlp_b0, mlp_W1, mlp_b1,
           hidden_W, hidden_b, nmf_W, nmf_b):
    user = user.astype(jnp.int32)
    item = item.astype(jnp.int32)
    gu, gi, mu, mi = _make_sc_gather()(
        user, item, gmf_user_table, gmf_item_table,
        mlp_user_table, mlp_item_table)
    fused, score = _make_tc_dense()(
        gu, gi, mu, mi,
        mlp_W0, mlp_b0.reshape(1, -1), mlp_W1, mlp_b1.reshape(1, -1),
        hidden_W, hidden_b.reshape(1, -1),
        nmf_W.reshape(1, -1), nmf_b.reshape(1, 1))
    return (score, fused)


# speedup vs baseline: 1.0146x; 1.0146x over previous
import functools

import jax
import jax.numpy as jnp
from jax import lax
from jax.experimental import pallas as pl
from jax.experimental.pallas import tpu as pltpu
from jax.experimental.pallas import tpu_sc as plsc

BATCH = 16384
DIM = 64
NC = 2
NS = 16
NW = NC * NS
PER_W = BATCH // NW


def _sc_gather_body(user_hbm, item_hbm, gu_t, gi_t, mu_t, mi_t,
                    gu_o, gi_o, mu_o, mi_o,
                    uidx, iidx, bufa, bufb, sema, semb):
    wid = lax.axis_index("s") * NC + lax.axis_index("c")
    base = wid * PER_W
    pltpu.sync_copy(user_hbm.at[pl.ds(base, PER_W)], uidx)
    pltpu.sync_copy(item_hbm.at[pl.ds(base, PER_W)], iidx)
    cpa = pltpu.async_copy(gu_t.at[uidx], bufa, sema)
    cpb = pltpu.async_copy(gi_t.at[iidx], bufb, semb)
    cpa.wait()
    pltpu.sync_copy(bufa, gu_o.at[pl.ds(base, PER_W)])
    cpa2 = pltpu.async_copy(mu_t.at[uidx], bufa, sema)
    cpb.wait()
    pltpu.sync_copy(bufb, gi_o.at[pl.ds(base, PER_W)])
    cpb2 = pltpu.async_copy(mi_t.at[iidx], bufb, semb)
    cpa2.wait()
    pltpu.sync_copy(bufa, mu_o.at[pl.ds(base, PER_W)])
    cpb2.wait()
    pltpu.sync_copy(bufb, mi_o.at[pl.ds(base, PER_W)])


def _make_sc_gather():
    mesh = plsc.VectorSubcoreMesh(core_axis_name="c", subcore_axis_name="s")
    row = jax.ShapeDtypeStruct((BATCH, DIM), jnp.float32)
    return pl.kernel(
        _sc_gather_body,
        out_type=[row, row, row, row],
        mesh=mesh,
        scratch_types=[
            pltpu.VMEM((PER_W,), jnp.int32),
            pltpu.VMEM((PER_W,), jnp.int32),
            pltpu.VMEM((PER_W, DIM), jnp.float32),
            pltpu.VMEM((PER_W, DIM), jnp.float32),
            pltpu.SemaphoreType.DMA,
            pltpu.SemaphoreType.DMA,
        ],
        compiler_params=pltpu.CompilerParams(use_tc_tiling_on_sc=False),
    )


BLK = 512


def _tc_dense_body(gu, gi, mu, mi, w0, b0, w1, b1, hw, hb, nw, nb,
                   fused_o, score_o):
    mlp_x = jnp.concatenate([mu[...], mi[...]], axis=1)
    h = jnp.maximum(jnp.dot(mlp_x, w0[...],
                            preferred_element_type=jnp.float32) + b0[...], 0.0)
    mlp_out = jnp.maximum(jnp.dot(h, w1[...],
                                  preferred_element_type=jnp.float32) + b1[...], 0.0)
    gmf = gu[...] * gi[...]
    fused_in = jnp.concatenate([0.5 * gmf, 0.5 * mlp_out], axis=1)
    fused = jnp.dot(fused_in, hw[...],
                    preferred_element_type=jnp.float32) + hb[...]
    fused_o[...] = fused
    score_o[...] = jnp.sum(fused * nw[...], axis=1, keepdims=True) + nb[...]


def _tc_dense_body1(gu, gi, mu, mi, w0, b0, w1, b1, hw, hb, nw, nb,
                    fused_o, score_o):
    mlp_x = jnp.concatenate([mu[...], mi[...]], axis=1)
    h = jnp.maximum(jnp.dot(mlp_x, w0[...],
                            preferred_element_type=jnp.float32) + b0[...], 0.0)
    mlp_out = jnp.maximum(jnp.dot(h, w1[...],
                                  preferred_element_type=jnp.float32) + b1[...], 0.0)
    gmf = gu[...] * gi[...]
    fused_in = jnp.concatenate([0.5 * gmf, 0.5 * mlp_out], axis=1)
    fused = jnp.dot(fused_in, hw[...],
                    preferred_element_type=jnp.float32) + hb[...]
    fused_o[...] = fused
    score_o[...] = jnp.sum(fused * nw[...], axis=1) + nb[0, 0]


def _make_tc_dense():
    grid = (BATCH // BLK,)
    blk_in = pl.BlockSpec((BLK, DIM), lambda i: (i, 0))
    full = lambda shape: pl.BlockSpec(shape, lambda i: (0, 0))
    return pl.pallas_call(
        _tc_dense_body1,
        grid=grid,
        in_specs=[
            blk_in, blk_in, blk_in, blk_in,
            full((2 * DIM, 2 * DIM)),
            full((1, 2 * DIM)),
            full((2 * DIM, DIM)),
            full((1, DIM)),
            full((2 * DIM, DIM)),
            full((1, DIM)),
            full((1, DIM)),
            full((1, 1)),
        ],
        out_specs=[
            pl.BlockSpec((BLK, DIM), lambda i: (i, 0)),
            pl.BlockSpec((BLK,), lambda i: (i,)),
        ],
        out_shape=[
            jax.ShapeDtypeStruct((BATCH, DIM), jnp.float32),
            jax.ShapeDtypeStruct((BATCH,), jnp.float32),
        ],
    )


def kernel(user, item, gmf_user_table, gmf_item_table, mlp_user_table,
           mlp_item_table, mlp_W0, mlp_b0, mlp_W1, mlp_b1,
           hidden_W, hidden_b, nmf_W, nmf_b):
    user = user.astype(jnp.int32)
    item = item.astype(jnp.int32)
    gu, gi, mu, mi = _make_sc_gather()(
        user, item, gmf_user_table, gmf_item_table,
        mlp_user_table, mlp_item_table)
    fused, score = _make_tc_dense()(
        gu, gi, mu, mi,
        mlp_W0, mlp_b0.reshape(1, -1), mlp_W1, mlp_b1.reshape(1, -1),
        hidden_W, hidden_b.reshape(1, -1),
        nmf_W.reshape(1, -1), nmf_b.reshape(1, 1))
    return (score.reshape(BATCH, 1), fused)


# speedup vs baseline: 1.2283x; 1.2106x over previous
import jax
import jax.numpy as jnp
from jax import lax
from jax.experimental import pallas as pl
from jax.experimental.pallas import tpu as pltpu
from jax.experimental.pallas import tpu_sc as plsc

BATCH = 16384
DIM = 64
WIDE = 2 * DIM
NC = 2
NS = 16
NW = NC * NS
PER_W = BATCH // NW
CHUNK = 128
NCH = PER_W // CHUNK


def _sc_gather_body(user_hbm, item_hbm, bu_t, bi_t, u_o, i_o,
                    idx0, idx1, buf0, buf1, sem0, sem1):
    wid = lax.axis_index("s") * NC + lax.axis_index("c")
    base = wid * PER_W
    tasks = [(user_hbm, bu_t, u_o, c) for c in range(NCH)] + \
            [(item_hbm, bi_t, i_o, c) for c in range(NCH)]
    idxs = (idx0, idx1)
    bufs = (buf0, buf1)
    sems = (sem0, sem1)

    def start(t):
        src_idx, tbl, _, c = tasks[t]
        pltpu.sync_copy(src_idx.at[pl.ds(base + c * CHUNK, CHUNK)],
                        idxs[t % 2])
        return pltpu.async_copy(tbl.at[idxs[t % 2]], bufs[t % 2], sems[t % 2])

    cp = start(0)
    for t in range(len(tasks)):
        nxt = None
        if t + 1 < len(tasks):
            nxt = start(t + 1)
        cp.wait()
        _, _, out, c = tasks[t]
        pltpu.sync_copy(bufs[t % 2], out.at[pl.ds(base + c * CHUNK, CHUNK)])
        cp = nxt


def _make_sc_gather():
    mesh = plsc.VectorSubcoreMesh(core_axis_name="c", subcore_axis_name="s")
    row = jax.ShapeDtypeStruct((BATCH, WIDE), jnp.float32)
    return pl.kernel(
        _sc_gather_body,
        out_type=[row, row],
        mesh=mesh,
        scratch_types=[
            pltpu.VMEM((CHUNK,), jnp.int32),
            pltpu.VMEM((CHUNK,), jnp.int32),
            pltpu.VMEM((CHUNK, WIDE), jnp.float32),
            pltpu.VMEM((CHUNK, WIDE), jnp.float32),
            pltpu.SemaphoreType.DMA,
            pltpu.SemaphoreType.DMA,
        ],
        compiler_params=pltpu.CompilerParams(use_tc_tiling_on_sc=True),
    )


BLK = 512


def _tc_dense_body(u, i, w0, b0, w1, b1, hw, hb, nw, nb,
                   fused_o, score_o):
    gu = u[:, :DIM]
    mu = u[:, DIM:]
    gi = i[:, :DIM]
    mi = i[:, DIM:]
    mlp_x = jnp.concatenate([mu, mi], axis=1)
    h = jnp.maximum(jnp.dot(mlp_x, w0[...],
                            preferred_element_type=jnp.float32) + b0[...], 0.0)
    mlp_out = jnp.maximum(jnp.dot(h, w1[...],
                                  preferred_element_type=jnp.float32) + b1[...], 0.0)
    gmf = gu * gi
    fused_in = jnp.concatenate([0.5 * gmf, 0.5 * mlp_out], axis=1)
    fused = jnp.dot(fused_in, hw[...],
                    preferred_element_type=jnp.float32) + hb[...]
    fused_o[...] = fused
    score_o[...] = jnp.sum(fused * nw[...], axis=1) + nb[0, 0]


def _make_tc_dense():
    grid = (BATCH // BLK,)
    blk_in = pl.BlockSpec((BLK, WIDE), lambda i: (i, 0))
    full = lambda shape: pl.BlockSpec(shape, lambda i: (0, 0))
    return pl.pallas_call(
        _tc_dense_body,
        grid=grid,
        in_specs=[
            blk_in, blk_in,
            full((WIDE, WIDE)),
            full((1, WIDE)),
            full((WIDE, DIM)),
            full((1, DIM)),
            full((WIDE, DIM)),
            full((1, DIM)),
            full((1, DIM)),
            full((1, 1)),
        ],
        out_specs=[
            pl.BlockSpec((BLK, DIM), lambda i: (i, 0)),
            pl.BlockSpec((BLK,), lambda i: (i,)),
        ],
        out_shape=[
            jax.ShapeDtypeStruct((BATCH, DIM), jnp.float32),
            jax.ShapeDtypeStruct((BATCH,), jnp.float32),
        ],
    )


def kernel(user, item, gmf_user_table, gmf_item_table, mlp_user_table,
           mlp_item_table, mlp_W0, mlp_b0, mlp_W1, mlp_b1,
           hidden_W, hidden_b, nmf_W, nmf_b):
    user = user.astype(jnp.int32)
    item = item.astype(jnp.int32)
    big_u = jnp.concatenate([gmf_user_table, mlp_user_table], axis=1)
    big_i = jnp.concatenate([gmf_item_table, mlp_item_table], axis=1)
    u_rows, i_rows = _make_sc_gather()(user, item, big_u, big_i)
    fused, score = _make_tc_dense()(
        u_rows, i_rows,
        mlp_W0, mlp_b0.reshape(1, -1), mlp_W1, mlp_b1.reshape(1, -1),
        hidden_W, hidden_b.reshape(1, -1),
        nmf_W.reshape(1, -1), nmf_b.reshape(1, 1))
    return (score.reshape(BATCH, 1), fused)
